# baseline (speedup 1.0000x reference)
import jax
import jax.numpy as jnp
from jax.experimental import pallas as pl
from jax.experimental.pallas import tpu as pltpu

_TB = 4608
_KC = 256
_SC = 8


def _vq_kernel(rep_hbm, cb_hbm, out_ref, rep_v, cb_v, sem_r, sem_c):
    i = pl.program_id(0)
    g = rep_v.shape[0]

    @pl.when(i == 0)
    def _():
        pltpu.make_async_copy(cb_hbm, cb_v, sem_c).start()
        pltpu.make_async_copy(cb_hbm, cb_v, sem_c).wait()

    pltpu.make_async_copy(rep_hbm.at[pl.ds(i * g, g)], rep_v, sem_r).start()
    pltpu.make_async_copy(rep_hbm.at[pl.ds(i * g, g)], rep_v, sem_r).wait()

    rep3 = rep_v[...]
    rep = rep3.reshape(-1, rep3.shape[2])
    cb = cb_v[...]
    k = cb.shape[0]
    d = rep.shape[1]
    m_len = rep3.shape[1]
    rep2 = rep * (-2.0)
    ones_row = jnp.ones((1, d), jnp.float32)
    a2 = jax.lax.dot_general(
        ones_row, rep * rep, (((1,), (1,)), ((), ())),
        preferred_element_type=jnp.float32)
    b2 = jnp.sum(cb * cb, axis=1, keepdims=True)
    runval = None
    runidx = None
    for c in range(k // _KC):
        abc = jax.lax.dot_general(
            cb[c * _KC:(c + 1) * _KC, :], rep2, (((1,), (1,)), ((), ())),
            preferred_element_type=jnp.float32)
        b2c = b2[c * _KC:(c + 1) * _KC, :]
        for r in range(_KC // _SC):
            rb = r * _SC
            s = b2c[rb:rb + _SC, :] + a2
            dist = s + abc[rb:rb + _SC, :]
            if runval is None:
                runval = dist
                runidx = jnp.zeros(dist.shape, jnp.int32)
            else:
                gi = c * (_KC // _SC) + r
                cond = dist < runval
                runval = jnp.minimum(dist, runval)
                runidx = jnp.where(cond, jnp.int32(gi), runidx)
    m = jnp.min(runval, axis=0, keepdims=True)
    srow = jax.lax.broadcasted_iota(jnp.int32, runval.shape, 0)
    kidx = runidx * _SC + srow
    cand = jnp.where(runval == m, kidx, jnp.int32(k))
    idx = jnp.min(cand, axis=0)
    for srw in range(idx.shape[0] // m_len):
        out_ref[srw, :] = idx[srw * m_len:(srw + 1) * m_len]


def kernel(rep, codebook):
    B, M, D = rep.shape
    K = codebook.shape[0]
    n = B * M
    nb = n // _TB
    g = _TB // M
    out = pl.pallas_call(
        _vq_kernel,
        grid=(nb,),
        in_specs=[
            pl.BlockSpec(memory_space=pl.ANY),
            pl.BlockSpec(memory_space=pl.ANY),
        ],
        out_specs=pl.BlockSpec((g, M), lambda i: (i, 0)),
        out_shape=jax.ShapeDtypeStruct((B, M), jnp.int32),
        scratch_shapes=[
            pltpu.VMEM((g, M, D), jnp.float32),
            pltpu.VMEM((K, D), jnp.float32),
            pltpu.SemaphoreType.DMA,
            pltpu.SemaphoreType.DMA,
        ],
        compiler_params=pltpu.CompilerParams(
            dimension_semantics=("arbitrary",),
        ),
    )(rep, codebook)
    return out


# speedup vs baseline: 1.2045x; 1.2045x over previous
import jax
import jax.numpy as jnp
from jax.experimental import pallas as pl
from jax.experimental.pallas import tpu as pltpu

_TB = 4608


_KC = 256
_SC = 8


def _vq_kernel(rep_ref, cb_ref, out_ref):
    rep3 = rep_ref[...]
    rep = rep3.reshape(-1, rep3.shape[2])
    cb = cb_ref[...]
    k = cb.shape[0]
    d = rep.shape[1]
    m_len = rep3.shape[1]
    rep2 = rep * (-2.0)
    ones_row = jnp.ones((1, d), jnp.float32)
    a2 = jax.lax.dot_general(
        ones_row, rep * rep, (((1,), (1,)), ((), ())),
        preferred_element_type=jnp.float32)
    b2 = jnp.sum(cb * cb, axis=1, keepdims=True)
    runval = None
    runidx = None
    for c in range(k // _KC):
        abc = jax.lax.dot_general(
            cb[c * _KC:(c + 1) * _KC, :], rep2, (((1,), (1,)), ((), ())),
            preferred_element_type=jnp.float32)
        b2c = b2[c * _KC:(c + 1) * _KC, :]
        for r in range(_KC // _SC):
            rb = r * _SC
            s = b2c[rb:rb + _SC, :] + a2
            dist = s + abc[rb:rb + _SC, :]
            if runval is None:
                runval = dist
                runidx = jnp.zeros(dist.shape, jnp.int32)
            else:
                gi = c * (_KC // _SC) + r
                cond = dist < runval
                runval = jnp.minimum(dist, runval)
                runidx = jnp.where(cond, jnp.int32(gi), runidx)
    m = jnp.min(runval, axis=0, keepdims=True)
    srow = jax.lax.broadcasted_iota(jnp.int32, runval.shape, 0)
    kidx = runidx * _SC + srow
    cand = jnp.where(runval == m, kidx, jnp.int32(k))
    idx = jnp.min(cand, axis=0)
    for srw in range(idx.shape[0] // m_len):
        out_ref[srw, :] = idx[srw * m_len:(srw + 1) * m_len]


def kernel(rep, codebook):
    B, M, D = rep.shape
    K = codebook.shape[0]
    n = B * M
    nb = n // _TB
    g = _TB // M
    out = pl.pallas_call(
        _vq_kernel,
        grid=(nb,),
        in_specs=[
            pl.BlockSpec((g, M, D), lambda i: (i, 0, 0)),
            pl.BlockSpec((K, D), lambda i: (0, 0)),
        ],
        out_specs=pl.BlockSpec((g, M), lambda i: (i, 0)),
        out_shape=jax.ShapeDtypeStruct((B, M), jnp.int32),
        compiler_params=pltpu.CompilerParams(
            dimension_semantics=("parallel",),
        ),
    )(rep, codebook)
    return out
